# Initial kernel scaffold; baseline (speedup 1.0000x reference)
import jax
import jax.numpy as jnp
from jax.experimental import pallas as pl


def kernel(insample_y, ln_gamma, ln_beta, Wg, bg, Wh, bh, Wout, bout):
    raise NotImplementedError("write your pallas kernel here")


# speedup vs baseline: 1.3078x; 1.3078x over previous
import jax
import jax.numpy as jnp
from jax.experimental import pallas as pl
from jax.experimental.pallas import tpu as pltpu

B = 8192
D = 512
E = 8
N_THETA = 608
BACKCAST = 512
BLK = 512


def _moe_dense_body(x_ref, g_ref, b_ref, wg_ref, bg_ref, wh_ref, bh_ref,
                    wout_ref, bout_ref, back_ref, fore_ref):
    x = x_ref[...]
    mu = jnp.mean(x, axis=-1, keepdims=True)
    xc = x - mu
    var = jnp.mean(xc * xc, axis=-1, keepdims=True)
    xn = xc * jax.lax.rsqrt(var + 1e-5) * g_ref[...] + b_ref[...]
    logits = jnp.dot(xn, wg_ref[...], preferred_element_type=jnp.float32)
    logits = logits + bg_ref[...]

    iota = jax.lax.broadcasted_iota(jnp.int32, logits.shape, 1)
    m1 = jnp.max(logits, axis=-1, keepdims=True)
    a1 = jnp.min(jnp.where(logits == m1, iota, E), axis=-1, keepdims=True)
    mask1 = iota == a1
    l2 = jnp.where(mask1, -jnp.inf, logits)
    m2 = jnp.max(l2, axis=-1, keepdims=True)
    a2 = jnp.min(jnp.where(l2 == m2, iota, E), axis=-1, keepdims=True)
    mask2 = iota == a2
    w1 = 1.0 / (1.0 + jnp.exp(m2 - m1))
    fw = jnp.where(mask1, w1, 0.0) + jnp.where(mask2, 1.0 - w1, 0.0)

    xb = x.astype(jnp.bfloat16)
    acc = jnp.zeros((BLK, N_THETA), jnp.float32)
    for e in range(E):
        h = jnp.dot(xb, wh_ref[e, 0], preferred_element_type=jnp.float32)
        h = h + bh_ref[e, 0]
        for i in range(1, 4):
            h = jnp.dot(h.astype(jnp.bfloat16), wh_ref[e, i],
                        preferred_element_type=jnp.float32) + bh_ref[e, i]
            h = jnp.maximum(h, 0.0)
        th = jnp.dot(h.astype(jnp.bfloat16), wout_ref[e],
                     preferred_element_type=jnp.float32) + bout_ref[e]
        acc = acc + fw[:, e:e + 1] * th
    back_ref[...] = acc[:, :BACKCAST]
    fore_ref[...] = acc[:, BACKCAST:]


def _full(shape):
    nd = len(shape)
    return pl.BlockSpec(shape, lambda i, _nd=nd: (0,) * _nd)


def kernel(insample_y, ln_gamma, ln_beta, Wg, bg, Wh, bh, Wout, bout):
    whb = Wh.astype(jnp.bfloat16)
    woutb = Wout.astype(jnp.bfloat16)
    back, fore = pl.pallas_call(
        _moe_dense_body,
        grid=(B // BLK,),
        in_specs=[
            pl.BlockSpec((BLK, D), lambda i: (i, 0)),
            _full((1, D)),
            _full((1, D)),
            _full((D, E)),
            _full((1, E)),
            _full((E, 4, D, D)),
            _full((E, 4, 1, D)),
            _full((E, D, N_THETA)),
            _full((E, 1, N_THETA)),
        ],
        out_specs=[
            pl.BlockSpec((BLK, BACKCAST), lambda i: (i, 0)),
            pl.BlockSpec((BLK, N_THETA - BACKCAST), lambda i: (i, 0)),
        ],
        out_shape=[
            jax.ShapeDtypeStruct((B, BACKCAST), jnp.float32),
            jax.ShapeDtypeStruct((B, N_THETA - BACKCAST), jnp.float32),
        ],
        compiler_params=pltpu.CompilerParams(
            vmem_limit_bytes=100 * 1024 * 1024,
        ),
    )(insample_y, ln_gamma.reshape(1, D), ln_beta.reshape(1, D), Wg,
      bg.reshape(1, E), whb, bh.reshape(E, 4, 1, D), woutb,
      bout.reshape(E, 1, N_THETA))
    return back, fore
